# baseline (speedup 1.0000x reference)
import functools

import jax
import jax.numpy as jnp
from jax import lax
from jax.experimental import pallas as pl
from jax.experimental.pallas import tpu as pltpu
from jax.experimental.pallas import tpu_sc as plsc

_VOCAB = 1000000
_EMB = 32
_TAGS = 50
_BATCH = 16384
_PACK = 256 // _EMB

_NC = 2
_NS = 16
_NW = _NC * _NS
_BPW = _BATCH // _NW
_CHUNK = 128
_NCHUNK = _BPW // _CHUNK

_sc_mesh = plsc.VectorSubcoreMesh(core_axis_name="c", subcore_axis_name="s")


@functools.partial(
    pl.kernel,
    mesh=_sc_mesh,
    out_type=jax.ShapeDtypeStruct((_BATCH, 128), jnp.int32),
    scratch_types=[
        pltpu.VMEM((_NCHUNK, _CHUNK), jnp.int32),
        pltpu.VMEM((_BPW, 128), jnp.int32),
        pltpu.SemaphoreType.DMA,
    ],
)
def _sc_gather(rows_hbm, table_hbm, out_hbm, idx_v, g_v, sem):
    wid = lax.axis_index("s") * _NC + lax.axis_index("c")
    base = wid * _BPW
    pltpu.sync_copy(rows_hbm.at[wid], idx_v)
    copies = []
    for c in range(_NCHUNK):
        copies.append(
            pltpu.async_copy(
                table_hbm.at[idx_v.at[c]],
                g_v.at[pl.ds(c * _CHUNK, _CHUNK)],
                sem,
            )
        )
    for cp in copies:
        cp.wait()
    pltpu.sync_copy(g_v, out_hbm.at[pl.ds(base, _BPW)])


_RL_BLK = 98304
_RL_SUB = _RL_BLK // _PACK
_RL_GRID = -(-_VOCAB // _RL_BLK)
_PROWS_PAD = _RL_GRID * _RL_SUB


def _relayout_body(tT_ref, o_ref):
    x = tT_ref[...].astype(jnp.bfloat16)
    halves = []
    for s in range(2):
        half = None
        for aa in range(4):
            a = 4 * s + aa
            ea = (jax.lax.broadcasted_iota(jnp.int32, (_EMB, 128), 0)
                  + 32 * aa ==
                  jax.lax.broadcasted_iota(jnp.int32, (_EMB, 128), 1)
                  ).astype(jnp.bfloat16)
            fa = lax.dot_general(
                x[:, a * _RL_SUB:(a + 1) * _RL_SUB], ea,
                (((0,), (0,)), ((), ())),
                preferred_element_type=jnp.float32,
                precision=lax.Precision.DEFAULT,
            )
            half = fa if half is None else half + fa
        halves.append(lax.bitcast_convert_type(
            half.astype(jnp.bfloat16), jnp.uint16).astype(jnp.uint32))
    o_ref[...] = lax.bitcast_convert_type(
        halves[0] | (halves[1] << 16), jnp.int32)


_tc_relayout = pl.pallas_call(
    _relayout_body,
    grid=(_RL_GRID,),
    in_specs=[pl.BlockSpec((_EMB, _RL_BLK), lambda i: (0, i))],
    out_specs=pl.BlockSpec((_RL_SUB, 128), lambda i: (i, 0)),
    out_shape=jax.ShapeDtypeStruct((_PROWS_PAD, 128), jnp.int32),
    compiler_params=pltpu.CompilerParams(
        dimension_semantics=("parallel",),
    ),
)


_TC_BLK = 2048


def _tc_body(g_ref, w8_ref, b_ref, m_ref, o_ref):
    gu = lax.bitcast_convert_type(g_ref[...], jnp.uint32)
    h0 = lax.bitcast_convert_type(
        (gu & 0xFFFF).astype(jnp.uint16), jnp.bfloat16)
    h1 = lax.bitcast_convert_type(
        (gu >> 16).astype(jnp.uint16), jnp.bfloat16)
    g = jnp.concatenate([h0, h1], axis=1)
    lane_grp = jax.lax.broadcasted_iota(jnp.int32, (_TC_BLK, _PACK * _EMB), 1)
    sel = (lane_grp // _EMB == m_ref[...]).astype(jnp.bfloat16)
    f = lax.dot_general(
        w8_ref[...], g * sel, (((0,), (1,)), ((), ())),
        preferred_element_type=jnp.float32,
        precision=lax.Precision.DEFAULT,
    ) + b_ref[...]
    mx = jnp.max(f, axis=0, keepdims=True)
    s = f - mx
    o_ref[...] = s - jnp.log(jnp.sum(jnp.exp(s), axis=0, keepdims=True))


_tc_tail = pl.pallas_call(
    _tc_body,
    grid=(_BATCH // _TC_BLK,),
    in_specs=[
        pl.BlockSpec((_TC_BLK, 128), lambda i: (i, 0)),
        pl.BlockSpec((_PACK * _EMB, _TAGS), lambda i: (0, 0)),
        pl.BlockSpec((_TAGS, 1), lambda i: (0, 0)),
        pl.BlockSpec((_TC_BLK, 1), lambda i: (i, 0)),
    ],
    out_specs=pl.BlockSpec((_TAGS, _TC_BLK), lambda i: (0, i)),
    out_shape=jax.ShapeDtypeStruct((_TAGS, _BATCH), jnp.float32),
    compiler_params=pltpu.CompilerParams(
        dimension_semantics=("parallel",),
    ),
)


def kernel(tokens, emb_table, W, b):
    tok = tokens.astype(jnp.int32)
    packed = _tc_relayout(emb_table.T)
    rows = (_RL_SUB * (tok // _RL_BLK) + (tok % _RL_BLK) % _RL_SUB).reshape(
        _NW, _NCHUNK, _CHUNK)
    g = _sc_gather(rows, packed)
    w8 = jnp.tile(W.T, (_PACK, 1)).astype(jnp.bfloat16)
    avec = ((tok % _RL_BLK) // _RL_SUB).reshape(_BATCH, 1)
    return _tc_tail(g, w8, b.reshape(_TAGS, 1), avec).T


# speedup vs baseline: 1.0244x; 1.0244x over previous
import functools

import jax
import jax.numpy as jnp
from jax import lax
from jax.experimental import pallas as pl
from jax.experimental.pallas import tpu as pltpu
from jax.experimental.pallas import tpu_sc as plsc

_VOCAB = 1000000
_EMB = 32
_TAGS = 50
_BATCH = 16384
_PACK = 256 // _EMB

_NC = 2
_NS = 16
_NW = _NC * _NS
_BPW = _BATCH // _NW
_CHUNK = 128
_NCHUNK = _BPW // _CHUNK

_sc_mesh = plsc.VectorSubcoreMesh(core_axis_name="c", subcore_axis_name="s")


@functools.partial(
    pl.kernel,
    mesh=_sc_mesh,
    out_type=jax.ShapeDtypeStruct((_BATCH, 128), jnp.int32),
    scratch_types=[
        pltpu.VMEM((_NCHUNK, _CHUNK), jnp.int32),
        pltpu.VMEM((_BPW, 128), jnp.int32),
        pltpu.SemaphoreType.DMA,
    ],
)
def _sc_gather(rows_hbm, table_hbm, out_hbm, idx_v, g_v, sem):
    wid = lax.axis_index("s") * _NC + lax.axis_index("c")
    base = wid * _BPW
    pltpu.sync_copy(rows_hbm.at[wid], idx_v)
    copies = []
    for c in range(_NCHUNK):
        copies.append(
            pltpu.async_copy(
                table_hbm.at[idx_v.at[c]],
                g_v.at[pl.ds(c * _CHUNK, _CHUNK)],
                sem,
            )
        )
    for cp in copies:
        cp.wait()
    pltpu.sync_copy(g_v, out_hbm.at[pl.ds(base, _BPW)])


_RL_BLK = 65536
_RL_SUB = _RL_BLK // _PACK
_RL_GRID = -(-_VOCAB // _RL_BLK)
_PROWS_PAD = _RL_GRID * _RL_SUB


def _relayout_body(tT_ref, o_ref):
    x = tT_ref[...].astype(jnp.bfloat16)
    halves = []
    for s in range(2):
        half = None
        for aa in range(4):
            a = 4 * s + aa
            ea = (jax.lax.broadcasted_iota(jnp.int32, (_EMB, 128), 0)
                  + 32 * aa ==
                  jax.lax.broadcasted_iota(jnp.int32, (_EMB, 128), 1)
                  ).astype(jnp.bfloat16)
            fa = lax.dot_general(
                x[:, a * _RL_SUB:(a + 1) * _RL_SUB], ea,
                (((0,), (0,)), ((), ())),
                preferred_element_type=jnp.float32,
                precision=lax.Precision.DEFAULT,
            )
            half = fa if half is None else half + fa
        halves.append(lax.bitcast_convert_type(
            half.astype(jnp.bfloat16), jnp.uint16).astype(jnp.uint32))
    o_ref[...] = lax.bitcast_convert_type(
        halves[0] | (halves[1] << 16), jnp.int32)


_tc_relayout = pl.pallas_call(
    _relayout_body,
    grid=(_RL_GRID,),
    in_specs=[pl.BlockSpec((_EMB, _RL_BLK), lambda i: (0, i))],
    out_specs=pl.BlockSpec((_RL_SUB, 128), lambda i: (i, 0)),
    out_shape=jax.ShapeDtypeStruct((_PROWS_PAD, 128), jnp.int32),
    compiler_params=pltpu.CompilerParams(
        dimension_semantics=("parallel",),
    ),
)


_TC_BLK = 2048


def _tc_body(g_ref, w8_ref, b_ref, m_ref, o_ref):
    gu = lax.bitcast_convert_type(g_ref[...], jnp.uint32)
    h0 = lax.bitcast_convert_type(
        (gu & 0xFFFF).astype(jnp.uint16), jnp.bfloat16)
    h1 = lax.bitcast_convert_type(
        (gu >> 16).astype(jnp.uint16), jnp.bfloat16)
    g = jnp.concatenate([h0, h1], axis=1)
    lane_grp = jax.lax.broadcasted_iota(jnp.int32, (_TC_BLK, _PACK * _EMB), 1)
    sel = (lane_grp // _EMB == m_ref[...]).astype(jnp.bfloat16)
    f = lax.dot_general(
        w8_ref[...], g * sel, (((0,), (1,)), ((), ())),
        preferred_element_type=jnp.float32,
        precision=lax.Precision.DEFAULT,
    ) + b_ref[...]
    mx = jnp.max(f, axis=0, keepdims=True)
    s = f - mx
    o_ref[...] = s - jnp.log(jnp.sum(jnp.exp(s), axis=0, keepdims=True))


_tc_tail = pl.pallas_call(
    _tc_body,
    grid=(_BATCH // _TC_BLK,),
    in_specs=[
        pl.BlockSpec((_TC_BLK, 128), lambda i: (i, 0)),
        pl.BlockSpec((_PACK * _EMB, _TAGS), lambda i: (0, 0)),
        pl.BlockSpec((_TAGS, 1), lambda i: (0, 0)),
        pl.BlockSpec((_TC_BLK, 1), lambda i: (i, 0)),
    ],
    out_specs=pl.BlockSpec((_TAGS, _TC_BLK), lambda i: (0, i)),
    out_shape=jax.ShapeDtypeStruct((_TAGS, _BATCH), jnp.float32),
    compiler_params=pltpu.CompilerParams(
        dimension_semantics=("parallel",),
    ),
)


def kernel(tokens, emb_table, W, b):
    tok = tokens.astype(jnp.int32)
    packed = _tc_relayout(emb_table.T)
    rows = (_RL_SUB * (tok >> 16) + (tok & (_RL_SUB - 1))).reshape(
        _NW, _NCHUNK, _CHUNK)
    g = _sc_gather(rows, packed)
    w8 = jnp.tile(W.T, (_PACK, 1)).astype(jnp.bfloat16)
    avec = ((tok >> 13) & 7).reshape(_BATCH, 1)
    return _tc_tail(g, w8, b.reshape(_TAGS, 1), avec).T
